# baseline (speedup 1.0000x reference)
import functools

import jax
import jax.numpy as jnp
from jax import lax
from jax.experimental import pallas as pl
from jax.experimental.pallas import tpu as pltpu
from jax.experimental.pallas import tpu_sc as plsc

NUM_EXPERTS = 16
TOP_K = 2
INPUT_DIM = 768
OUTPUT_DIM = 256
TOKEN_BLOCK = 512
N_TOKENS = 4096

_NC = 2
_NS = 16
_ROWS_PER_W = N_TOKENS // (_NC * _NS)


def _gate_logits_kernel(x_ref, wg_ref, bg_ref, o_ref):
    o_ref[...] = jax.lax.dot_general(
        x_ref[...], wg_ref[...], (((1,), (1,)), ((), ())),
        preferred_element_type=jnp.float32) + bg_ref[...]


def _route_sc_kernel(logits_hbm, cw_hbm, buf, obuf):
    wid = lax.axis_index("s") * _NC + lax.axis_index("c")
    base = wid * _ROWS_PER_W
    pltpu.sync_copy(logits_hbm.at[pl.ds(base, _ROWS_PER_W)], buf)
    e_iota = lax.iota(jnp.int32, NUM_EXPERTS)
    neg_inf = jnp.float32(-jnp.inf)

    def allreduce(v, op):
        for s in (8, 4, 2, 1):
            shuf = v.at[e_iota ^ s].get(mode="promise_in_bounds",
                                        unique_indices=True)
            v = op(v, shuf)
        return v

    def body(i, _):
        row = buf[i]
        m = allreduce(row, jnp.maximum)
        e = jnp.exp(row - m)
        w = e / allreduce(e, jnp.add)
        big = jnp.int32(NUM_EXPERTS)
        v1 = allreduce(w, jnp.maximum)
        i1 = allreduce(jnp.where(w == v1, e_iota, big), jnp.minimum)
        w2 = jnp.where(e_iota == i1, neg_inf, w)
        v2 = allreduce(w2, jnp.maximum)
        i2 = allreduce(jnp.where(w2 == v2, e_iota, big), jnp.minimum)
        obuf[i] = (jnp.where(e_iota == i1, v1, 0.0)
                   + jnp.where(e_iota == i2, v2, 0.0))
        return 0

    lax.fori_loop(0, _ROWS_PER_W, body, 0)
    pltpu.sync_copy(obuf, cw_hbm.at[pl.ds(base, _ROWS_PER_W)])


_route_sc = functools.partial(
    pl.kernel,
    mesh=plsc.VectorSubcoreMesh(core_axis_name="c", subcore_axis_name="s"),
    out_type=jax.ShapeDtypeStruct((N_TOKENS, NUM_EXPERTS), jnp.float32),
    scratch_types=[
        pltpu.VMEM((_ROWS_PER_W, NUM_EXPERTS), jnp.float32),
        pltpu.VMEM((_ROWS_PER_W, NUM_EXPERTS), jnp.float32),
    ],
)(_route_sc_kernel)


def _moe_block_kernel(x_ref, cw_ref, we_ref, be_ref, o_ref):
    x = x_ref[...]
    cw = cw_ref[...]
    acc = jnp.zeros((x.shape[0], OUTPUT_DIM), jnp.float32)
    for e in range(NUM_EXPERTS):
        ye = jax.lax.dot_general(
            x, we_ref[e], (((1,), (1,)), ((), ())),
            preferred_element_type=jnp.float32)
        acc = acc + cw[:, e][:, None] * (ye + be_ref[e][None, :])
    o_ref[...] = acc


@jax.jit
def kernel(x, Wg, bg, We, be):
    bs, L, d = x.shape
    n_tokens = bs * L
    xf = x.reshape(n_tokens, d)

    logits = pl.pallas_call(
        _gate_logits_kernel,
        grid=(n_tokens // 1024,),
        in_specs=[
            pl.BlockSpec((1024, d), lambda i: (i, 0)),
            pl.BlockSpec((NUM_EXPERTS, d), lambda i: (0, 0)),
            pl.BlockSpec((1, NUM_EXPERTS), lambda i: (0, 0)),
        ],
        out_specs=pl.BlockSpec((1024, NUM_EXPERTS), lambda i: (i, 0)),
        out_shape=jax.ShapeDtypeStruct((n_tokens, NUM_EXPERTS), jnp.float32),
    )(xf, Wg, bg.reshape(1, NUM_EXPERTS))

    cw = _route_sc(logits)

    out = pl.pallas_call(
        _moe_block_kernel,
        grid=(n_tokens // TOKEN_BLOCK,),
        in_specs=[
            pl.BlockSpec((TOKEN_BLOCK, d), lambda i: (i, 0)),
            pl.BlockSpec((TOKEN_BLOCK, NUM_EXPERTS), lambda i: (i, 0)),
            pl.BlockSpec((NUM_EXPERTS, OUTPUT_DIM, d), lambda i: (0, 0, 0)),
            pl.BlockSpec((NUM_EXPERTS, OUTPUT_DIM), lambda i: (0, 0)),
        ],
        out_specs=pl.BlockSpec((TOKEN_BLOCK, OUTPUT_DIM), lambda i: (i, 0)),
        out_shape=jax.ShapeDtypeStruct((n_tokens, OUTPUT_DIM), jnp.float32),
    )(xf, cw, We, be)
    return out.reshape(bs, L, OUTPUT_DIM)


# speedup vs baseline: 1.6986x; 1.6986x over previous
import jax
import jax.numpy as jnp
from jax.experimental import pallas as pl

NUM_EXPERTS = 16
TOP_K = 2
INPUT_DIM = 768
OUTPUT_DIM = 256
TOKEN_BLOCK = 512


def _moe_block_kernel(x_ref, wg_ref, bg_ref, we_ref, be_ref, o_ref):
    x = x_ref[...]
    logits = jax.lax.dot_general(
        x, wg_ref[...], (((1,), (1,)), ((), ())),
        preferred_element_type=jnp.float32) + bg_ref[...]
    w = jax.nn.softmax(logits, axis=-1)
    e_iota = jax.lax.broadcasted_iota(jnp.int32, w.shape, 1)
    i1 = jnp.argmax(w, axis=-1)[:, None]
    v1 = jnp.max(w, axis=-1)[:, None]
    w2 = jnp.where(e_iota == i1, -jnp.inf, w)
    i2 = jnp.argmax(w2, axis=-1)[:, None]
    v2 = jnp.max(w2, axis=-1)[:, None]
    cw = (jnp.where(e_iota == i1, v1, 0.0)
          + jnp.where(e_iota == i2, v2, 0.0))

    xb = x.astype(jnp.bfloat16)
    w2d = we_ref[...].reshape(
        NUM_EXPERTS * OUTPUT_DIM, INPUT_DIM).astype(jnp.bfloat16)
    y_all = jax.lax.dot_general(
        xb, w2d, (((1,), (1,)), ((), ())),
        preferred_element_type=jnp.float32)
    acc = jnp.zeros((x.shape[0], OUTPUT_DIM), jnp.float32)
    for e in range(NUM_EXPERTS):
        ye = y_all[:, e * OUTPUT_DIM:(e + 1) * OUTPUT_DIM]
        acc = acc + cw[:, e][:, None] * (ye + be_ref[e][None, :])
    o_ref[...] = acc


@jax.jit
def kernel(x, Wg, bg, We, be):
    bs, L, d = x.shape
    n_tokens = bs * L
    xf = x.reshape(n_tokens, d)
    grid = (n_tokens // TOKEN_BLOCK,)
    out = pl.pallas_call(
        _moe_block_kernel,
        grid=grid,
        in_specs=[
            pl.BlockSpec((TOKEN_BLOCK, d), lambda i: (i, 0)),
            pl.BlockSpec((NUM_EXPERTS, d), lambda i: (0, 0)),
            pl.BlockSpec((1, NUM_EXPERTS), lambda i: (0, 0)),
            pl.BlockSpec((NUM_EXPERTS, OUTPUT_DIM, d), lambda i: (0, 0, 0)),
            pl.BlockSpec((NUM_EXPERTS, OUTPUT_DIM), lambda i: (0, 0)),
        ],
        out_specs=pl.BlockSpec((TOKEN_BLOCK, OUTPUT_DIM), lambda i: (i, 0)),
        out_shape=jax.ShapeDtypeStruct((n_tokens, OUTPUT_DIM), jnp.float32),
    )(xf, Wg, bg.reshape(1, NUM_EXPERTS), We, be)
    return out.reshape(bs, L, OUTPUT_DIM)
